# baseline (speedup 1.0000x reference)
import functools

import jax
import jax.numpy as jnp
from jax import lax
from jax.experimental import pallas as pl
from jax.experimental.pallas import tpu as pltpu
from jax.experimental.pallas import tpu_sc as plsc

B = 1024
V = 100000
S = 128
NC, NS, L = 2, 16, 16
NW = NC * NS
RPW = B // NW
LANES = 128
NBUF = 2
CB = 2

_LOG1P = (
    1.4720650114430356e-06,
    0.9998476974962223,
    -0.49737321615786034,
    0.31574731675770634,
    -0.19035433673267907,
    0.08269123711113481,
    -0.01741407752417672,
)


def _sc_loss_partials(input_t, target, samples):
    mesh = plsc.VectorSubcoreMesh(core_axis_name="c", subcore_axis_name="s")

    @functools.partial(
        pl.kernel,
        mesh=mesh,
        out_type=jax.ShapeDtypeStruct((NW * L,), jnp.float32),
        scratch_types=[
            pltpu.VMEM((RPW, S), jnp.int32),
            pltpu.VMEM((RPW * S,), jnp.int32),
            pltpu.VMEM((RPW,), jnp.int32),
            pltpu.VMEM((NBUF, CB * S, LANES), jnp.float32),
            pltpu.VMEM((RPW, LANES), jnp.float32),
            pltpu.VMEM((RPW,), jnp.float32),
            pltpu.SemaphoreType.DMA,
        ],
        compiler_params=pltpu.CompilerParams(needs_layout_passes=False),
    )
    def k(inp_hbm, tgt_hbm, smp_hbm, pout_hbm,
          idx_v, fidx_v, tid_v, lines_v, tlines_v, tsc_v, sem):
        wid = lax.axis_index("s") * NC + lax.axis_index("c")
        base = wid * RPW
        col0 = pl.multiple_of((wid // 4) * LANES, LANES)
        lane0 = base - col0

        pltpu.sync_copy(smp_hbm.at[pl.ds(base, RPW)], idx_v)
        pltpu.sync_copy(tgt_hbm.at[pl.ds(base, RPW)], tid_v)

        tcopy = pltpu.async_copy(
            inp_hbm.at[tid_v, pl.ds(col0, LANES)], tlines_v, sem
        )

        def flat(r, _):
            fo = pl.multiple_of(r * S, S)
            for kk in range(S // L):
                fidx_v[pl.ds(fo + kk * L, L)] = idx_v[r, pl.ds(kk * L, L)]
            return 0

        lax.fori_loop(0, RPW, flat, 0)

        def chunk_copy(cc):
            return pltpu.make_async_copy(
                inp_hbm.at[
                    fidx_v.at[pl.ds(pl.multiple_of(cc * CB * S, S), CB * S)],
                    pl.ds(col0, LANES),
                ],
                lines_v.at[cc % NBUF], sem,
            )

        for cc in range(NBUF):
            chunk_copy(cc).start()

        tcopy.wait()
        for kk in range(RPW // L):
            rows = lax.iota(jnp.int32, L) + kk * L
            x = plsc.load_gather(tlines_v, [rows, lane0 + rows])
            tsc_v[pl.ds(kk * L, L)] = x

        def step(cc, acc):
            chunk_copy(cc).wait()

            @pl.when(cc + NBUF < RPW // CB)
            def _():
                chunk_copy(cc + NBUF).start()

            buf = jnp.full((L,), cc % NBUF, jnp.int32)
            for b in range(CB):
                r = cc * CB + b
                tsp = plsc.load_gather(tsc_v, [jnp.full((L,), r, jnp.int32)])
                lvec = jnp.full((L,), lane0 + r, jnp.int32)
                for kk in range(S // L):
                    rows = lax.iota(jnp.int32, L) + b * S + kk * L
                    sc = plsc.load_gather(lines_v, [buf, rows, lvec])
                    x = tsp - sc
                    e = jnp.exp(-jnp.abs(x))
                    p = jnp.float32(_LOG1P[-1])
                    for c in _LOG1P[-2::-1]:
                        p = p * e + jnp.float32(c)
                    acc = acc + (jnp.minimum(x, 0.0) - p)
            return acc

        acc = lax.fori_loop(
            0, RPW // CB, step, jnp.zeros((L,), jnp.float32)
        )
        tlines_v[0, pl.ds(0, L)] = acc
        pltpu.sync_copy(
            tlines_v.at[0, pl.ds(0, L)],
            pout_hbm.at[pl.ds(wid * L, L)],
        )

    return k(input_t, target, samples)


def _tc_final(partials):
    def body(p_ref, o_ref):
        o_ref[0, 0] = -jnp.sum(p_ref[...]) * (1.0 / (B * S))

    return pl.pallas_call(
        body,
        out_shape=jax.ShapeDtypeStruct((1, 1), jnp.float32),
        out_specs=pl.BlockSpec(memory_space=pltpu.SMEM),
    )(partials)


def kernel(input, target, samples):
    p = _sc_loss_partials(
        input.T, target.astype(jnp.int32), samples.astype(jnp.int32)
    )
    return _tc_final(p.reshape(NW // 8, 8 * L))[0, 0]
